# baseline (speedup 1.0000x reference)
import functools

import jax
import jax.numpy as jnp
from jax import lax
from jax.experimental import pallas as pl
from jax.experimental.pallas import tpu as pltpu
from jax.experimental.pallas import tpu_sc as plsc

N = 100000
E = 200000
IN = 128
H = 4
DH = 32
F = H * DH

NC = 2
NT = 16
W16 = 16

EPT = 13312
TROWS = EPT // 128
E_PAD = EPT * NT
EROWS = E_PAD // 128

NK = 8
CS = 6272
CR = NK * CS
NOUT = NC * CR
NPAD = NOUT
WB = CS // NT
BB = 128
SR = 8
NSTRIP = TROWS // SR

_SPLAT_DNUMS = lax.GatherDimensionNumbers(
    offset_dims=(), collapsed_slice_dims=(0,), start_index_map=(0,))


def _splat(v, h):
    idx = jnp.full((16, 1), h, dtype=jnp.int32)
    return lax.gather(v, idx, _SPLAT_DNUMS, (1,),
                      mode=lax.GatherScatterMode.PROMISE_IN_BOUNDS)


def _expand_attn(a):
    eye = jnp.eye(W16, dtype=a.dtype)[:H]
    return (a[:, :, None] * eye[:, None, :]).reshape(IN, W16)


def _dense(x, Ws, ALs, ARs):
    R = 1000

    def body(x_ref, w_ref, al_ref, ar_ref,
             f0, f1, f2, el0, er0, el1, er1, el2, er2):
        xb = x_ref[...]
        outs = ((f0, el0, er0), (f1, el1, er1), (f2, el2, er2))
        for r in range(3):
            fr, elr, err = outs[r]
            feat = jnp.dot(xb, w_ref[r], preferred_element_type=jnp.float32)
            fr[...] = feat
            elr[...] = jnp.dot(feat, al_ref[r],
                               preferred_element_type=jnp.float32)
            err[...] = jnp.dot(feat, ar_ref[r],
                               preferred_element_type=jnp.float32)

    return pl.pallas_call(
        body,
        grid=(N // R,),
        in_specs=[
            pl.BlockSpec((R, IN), lambda i: (i, 0)),
            pl.BlockSpec((3, IN, F), lambda i: (0, 0, 0)),
            pl.BlockSpec((3, IN, W16), lambda i: (0, 0, 0)),
            pl.BlockSpec((3, IN, W16), lambda i: (0, 0, 0)),
        ],
        out_specs=[pl.BlockSpec((R, F), lambda i: (i, 0))] * 3
        + [pl.BlockSpec((R, W16), lambda i: (i, 0))] * 6,
        out_shape=[jax.ShapeDtypeStruct((N, F), jnp.float32)] * 3
        + [jax.ShapeDtypeStruct((N, W16), jnp.float32)] * 6,
    )(x, Ws, ALs, ARs)


def _lrelu(v):
    return jnp.where(v >= 0.0, v, v * jnp.float32(0.2))


def _denoms(elps, erps, edges, z16):
    mesh = plsc.VectorSubcoreMesh(core_axis_name="c", subcore_axis_name="s")

    @functools.partial(
        pl.kernel,
        out_type=[jax.ShapeDtypeStruct((NPAD, W16), jnp.float32)] * 3,
        mesh=mesh,
        compiler_params=pltpu.CompilerParams(use_tc_tiling_on_sc=False, needs_layout_passes=False),
        scratch_types=[
            pltpu.VMEM_SHARED((NPAD, W16), jnp.float32),
            pltpu.VMEM((SR, 128), jnp.int32),
            pltpu.VMEM((SR, 128), jnp.int32),
            pltpu.VMEM((128, W16), jnp.float32),
            pltpu.VMEM((128, W16), jnp.float32),
            pltpu.VMEM((128, W16), jnp.float32),
            pltpu.SemaphoreType.DMA,
            pltpu.SemaphoreType.DMA,
        ],
    )
    def run(el0, el1, el2, er0, er1, er2, z_hbm,
            s0, d0, s1, d1, s2, d2,
            den0, den1, den2,
            den_s, src_v, dst_v, elg, erg, eev, sem_a, sem_b):
        cid = lax.axis_index("c")
        tid = lax.axis_index("s")
        zrows = NPAD // NT

        def do_rel(el, er, s2d, d2d, den_o):
            pltpu.sync_copy(z_hbm, den_s.at[pl.ds(tid * zrows, zrows)])
            plsc.subcore_barrier()

            def strip(st, carry):
                row0 = tid * TROWS + st * SR
                pltpu.sync_copy(s2d.at[pl.ds(row0, SR)], src_v)
                pltpu.sync_copy(d2d.at[pl.ds(row0, SR)], dst_v)

                def jloop(j, c1):
                    ca = pltpu.async_copy(el.at[src_v.at[j]], elg, sem_a)
                    cb = pltpu.async_copy(er.at[dst_v.at[j]], erg, sem_b)
                    ca.wait()
                    cb.wait()

                    def vloop(i, c2):
                        ee = jnp.exp(_lrelu(elg[i, :] + erg[i, :]))
                        eev[i, :] = ee
                        return c2

                    lax.fori_loop(0, 128, vloop, 0)
                    pltpu.sync_copy(eev, den_s.at[dst_v.at[j]], add=True)
                    return c1

                lax.fori_loop(0, SR, jloop, 0)
                return carry

            lax.fori_loop(0, NSTRIP, strip, 0)
            plsc.subcore_barrier()
            pltpu.sync_copy(den_s.at[pl.ds(tid * zrows, zrows)],
                            den_o.at[pl.ds(tid * zrows, zrows)])

        rel_args = (
            (el0, er0, (s0, d0), den0),
            (el1, er1, (s1, d1), den1),
            (el2, er2, (s2, d2), den2),
        )
        for r in range(3):
            el, er, (s2d, d2d), den_o = rel_args[r]
            pl.when(cid == r % 2)(
                functools.partial(do_rel, el, er, s2d, d2d, den_o))

    return run(*elps, *erps, z16, *edges)


def _messages(feats, elps, erps, dens, edges, binit):
    mesh = plsc.VectorSubcoreMesh(core_axis_name="c", subcore_axis_name="s")

    @functools.partial(
        pl.kernel,
        out_type=jax.ShapeDtypeStruct((NOUT, F), jnp.float32),
        mesh=mesh,
        compiler_params=pltpu.CompilerParams(use_tc_tiling_on_sc=False, needs_layout_passes=False),
        scratch_types=[
            pltpu.VMEM_SHARED((CS, F), jnp.float32),
            pltpu.VMEM((SR, 128), jnp.int32),
            pltpu.VMEM((SR, 128), jnp.int32),
            pltpu.VMEM((EPT + BB,), jnp.int32),
            pltpu.VMEM((BB, F), jnp.float32),
            pltpu.VMEM((BB, W16), jnp.float32),
            pltpu.VMEM((BB, W16), jnp.float32),
            pltpu.VMEM((BB, W16), jnp.float32),
            pltpu.VMEM((1, 128), jnp.int32),
            pltpu.VMEM((1, 128), jnp.int32),
            pltpu.VMEM((1, 128), jnp.int32),
            pltpu.SemaphoreType.DMA,
        ],
    )
    def run(f0, f1, f2, el0, el1, el2, er0, er1, er2,
            den0, den1, den2, s0, d0, s1, d1, s2, d2, binit_hbm,
            out,
            acc, src_v, dst_v, sel_sd, fbuf,
            elg, erg, deng, isrc2, idst2a, idst2r, sem):
        cid = lax.axis_index("c")
        tid = lax.axis_index("s")
        rels = ((f0, el0, er0, den0, s0, d0),
                (f1, el1, er1, den1, s1, d1),
                (f2, el2, er2, den2, s2, d2))
        zero16i = jnp.zeros((16,), jnp.int32)

        def chunk(k, carry):
            lo = cid * CR + k * CS
            pltpu.sync_copy(binit_hbm, acc.at[pl.ds(tid * WB, WB)])
            plsc.subcore_barrier()
            for r in range(3):
                feat, elp, erp, den, s2d, d2d = rels[r]

                def strip(st, cur):
                    row0 = tid * TROWS + st * SR
                    pltpu.sync_copy(s2d.at[pl.ds(row0, SR)], src_v)
                    pltpu.sync_copy(d2d.at[pl.ds(row0, SR)], dst_v)

                    def scan_body(i, cur_v):
                        row = i >> 3
                        col = (i & 7) << 4
                        s = src_v[row, pl.ds(col, 16)]
                        d = dst_v[row, pl.ds(col, 16)]
                        m = (d >= lo) & (d < lo + CS)
                        mi = m.astype(jnp.int32)
                        cs = plsc.cumsum(mi)
                        pos = cur_v + cs - 1
                        packed = s | ((d - lo) << 17)
                        plsc.store_scatter(sel_sd, [pos], packed, mask=m)
                        return cur_v + _splat(cs, 15)

                    return lax.fori_loop(0, SR * 8, scan_body, cur)

                cur_v = lax.fori_loop(0, NSTRIP, strip,
                                      jnp.zeros((16,), jnp.int32))
                cnt = jnp.sum(cur_v) >> 4
                for q in range(BB // 16):
                    sel_sd[pl.ds(cnt + q * 16, 16)] = zero16i
                nb = (cnt + (BB - 1)) >> 7

                def batch_body(b, carry2):
                    bo = b * BB
                    for q in range(BB // 16):
                        pv = sel_sd[pl.ds(bo + q * 16, 16)]
                        dv = pv >> 17
                        isrc2[0, pl.ds(q * 16, 16)] = pv & 0x1FFFF
                        idst2r[0, pl.ds(q * 16, 16)] = dv
                        idst2a[0, pl.ds(q * 16, 16)] = dv + lo
                    c1 = pltpu.async_copy(feat.at[isrc2.at[0]], fbuf, sem)
                    c2 = pltpu.async_copy(elp.at[isrc2.at[0]], elg, sem)
                    c3 = pltpu.async_copy(erp.at[idst2a.at[0]], erg, sem)
                    c4 = pltpu.async_copy(den.at[idst2a.at[0]], deng, sem)
                    c1.wait()
                    c2.wait()
                    c3.wait()
                    c4.wait()

                    def mloop(e, c2_):
                        ee = jnp.exp(_lrelu(elg[e, :] + erg[e, :]))
                        a = ee / jnp.maximum(deng[e, :], jnp.float32(1e-9))
                        valid = ((bo + e) < cnt).astype(jnp.float32)
                        a = a * valid
                        for h in range(H):
                            sp = _splat(a, h)
                            for p2 in range(2):
                                p = h * 2 + p2
                                fv = fbuf[e, pl.ds(p * 16, 16)]
                                fbuf[e, pl.ds(p * 16, 16)] = fv * sp
                        return c2_

                    lax.fori_loop(0, BB, mloop, 0)
                    pltpu.sync_copy(fbuf, acc.at[idst2r.at[0]], add=True)
                    return carry2

                lax.fori_loop(0, nb, batch_body, 0)
            plsc.subcore_barrier()
            pltpu.sync_copy(acc.at[pl.ds(tid * WB, WB)],
                            out.at[pl.ds(lo + tid * WB, WB)])
            return carry

        lax.fori_loop(0, NK, chunk, 0)

    return run(*feats, *elps, *erps, *dens, *edges, binit)


def _pad_nodes(a):
    return jnp.pad(a, ((0, NPAD - N), (0, 0)))


def _pad_edges(ei):
    s = jnp.concatenate([ei[0], jnp.zeros((E_PAD - E,), jnp.int32)])
    d = jnp.concatenate([ei[1], jnp.full((E_PAD - E,), N, jnp.int32)])
    return s.reshape(EROWS, 128), d.reshape(EROWS, 128)


def kernel(x, edge_index_r0, edge_index_r1, edge_index_r2,
           W_r0, attn_l_r0, attn_r_r0, bias_r0,
           W_r1, attn_l_r1, attn_r_r1, bias_r1,
           W_r2, attn_l_r2, attn_r_r2, bias_r2):
    Ws = jnp.stack([W_r0, W_r1, W_r2])
    ALs = jnp.stack([_expand_attn(attn_l_r0), _expand_attn(attn_l_r1),
                     _expand_attn(attn_l_r2)])
    ARs = jnp.stack([_expand_attn(attn_r_r0), _expand_attn(attn_r_r1),
                     _expand_attn(attn_r_r2)])

    f0, f1, f2, el0, er0, el1, er1, el2, er2 = _dense(x, Ws, ALs, ARs)
    elps = [_pad_nodes(el0), _pad_nodes(el1), _pad_nodes(el2)]
    erps = [_pad_nodes(er0), _pad_nodes(er1), _pad_nodes(er2)]

    s0, d0 = _pad_edges(edge_index_r0)
    s1, d1 = _pad_edges(edge_index_r1)
    s2, d2 = _pad_edges(edge_index_r2)
    edges = (s0, d0, s1, d1, s2, d2)

    z16 = jnp.zeros((NPAD // NT, W16), jnp.float32)
    den0, den1, den2 = _denoms(elps, erps, edges, z16)

    bias_sum = (bias_r0 + bias_r1 + bias_r2).astype(jnp.float32)
    binit = jnp.tile(bias_sum.reshape(1, F), (WB, 1))
    out_full = _messages((f0, f1, f2), elps, erps, (den0, den1, den2),
                         edges, binit)
    return out_full[:N]


# speedup vs baseline: 1.0047x; 1.0047x over previous
import functools

import jax
import jax.numpy as jnp
from jax import lax
from jax.experimental import pallas as pl
from jax.experimental.pallas import tpu as pltpu
from jax.experimental.pallas import tpu_sc as plsc

N = 100000
E = 200000
IN = 128
H = 4
DH = 32
F = H * DH

NC = 2
NT = 16
W16 = 16

EPT = 13312
TROWS = EPT // 128
E_PAD = EPT * NT
EROWS = E_PAD // 128

NK = 8
CS = 6272
CR = NK * CS
NOUT = NC * CR
NPAD = NOUT
WB = CS // NT
BB = 128
SR = 8
NSTRIP = TROWS // SR

_SPLAT_DNUMS = lax.GatherDimensionNumbers(
    offset_dims=(), collapsed_slice_dims=(0,), start_index_map=(0,))


def _splat(v, h):
    idx = jnp.full((16, 1), h, dtype=jnp.int32)
    return lax.gather(v, idx, _SPLAT_DNUMS, (1,),
                      mode=lax.GatherScatterMode.PROMISE_IN_BOUNDS)


def _expand_attn(a):
    eye = jnp.eye(W16, dtype=a.dtype)[:H]
    return (a[:, :, None] * eye[:, None, :]).reshape(IN, W16)


def _dense(x, Ws, ALs, ARs):
    R = 1000

    def body(x_ref, w_ref, al_ref, ar_ref,
             f0, f1, f2, el0, er0, el1, er1, el2, er2):
        xb = x_ref[...]
        outs = ((f0, el0, er0), (f1, el1, er1), (f2, el2, er2))
        for r in range(3):
            fr, elr, err = outs[r]
            feat = jnp.dot(xb, w_ref[r], preferred_element_type=jnp.float32)
            fr[...] = feat
            elr[...] = jnp.dot(feat, al_ref[r],
                               preferred_element_type=jnp.float32)
            err[...] = jnp.dot(feat, ar_ref[r],
                               preferred_element_type=jnp.float32)

    return pl.pallas_call(
        body,
        grid=(N // R,),
        in_specs=[
            pl.BlockSpec((R, IN), lambda i: (i, 0)),
            pl.BlockSpec((3, IN, F), lambda i: (0, 0, 0)),
            pl.BlockSpec((3, IN, W16), lambda i: (0, 0, 0)),
            pl.BlockSpec((3, IN, W16), lambda i: (0, 0, 0)),
        ],
        out_specs=[pl.BlockSpec((R, F), lambda i: (i, 0))] * 3
        + [pl.BlockSpec((R, W16), lambda i: (i, 0))] * 6,
        out_shape=[jax.ShapeDtypeStruct((N, F), jnp.float32)] * 3
        + [jax.ShapeDtypeStruct((N, W16), jnp.float32)] * 6,
    )(x, Ws, ALs, ARs)


def _lrelu(v):
    return jnp.where(v >= 0.0, v, v * jnp.float32(0.2))


def _denoms(elps, erps, edges, z16):
    mesh = plsc.VectorSubcoreMesh(core_axis_name="c", subcore_axis_name="s")

    @functools.partial(
        pl.kernel,
        out_type=[jax.ShapeDtypeStruct((NPAD, W16), jnp.float32)] * 3,
        mesh=mesh,
        compiler_params=pltpu.CompilerParams(use_tc_tiling_on_sc=False, needs_layout_passes=False),
        scratch_types=[
            pltpu.VMEM_SHARED((NPAD, W16), jnp.float32),
            pltpu.VMEM((SR, 128), jnp.int32),
            pltpu.VMEM((SR, 128), jnp.int32),
            pltpu.VMEM((128, W16), jnp.float32),
            pltpu.VMEM((128, W16), jnp.float32),
            pltpu.VMEM((128, W16), jnp.float32),
            pltpu.SemaphoreType.DMA,
            pltpu.SemaphoreType.DMA,
        ],
    )
    def run(el0, el1, el2, er0, er1, er2, z_hbm,
            s0, d0, s1, d1, s2, d2,
            den0, den1, den2,
            den_s, src_v, dst_v, elg, erg, eev, sem_a, sem_b):
        cid = lax.axis_index("c")
        tid = lax.axis_index("s")
        zrows = NPAD // NT

        def do_rel(el, er, s2d, d2d, den_o):
            pltpu.sync_copy(z_hbm, den_s.at[pl.ds(tid * zrows, zrows)])
            plsc.subcore_barrier()

            def strip(st, carry):
                row0 = tid * TROWS + st * SR
                pltpu.sync_copy(s2d.at[pl.ds(row0, SR)], src_v)
                pltpu.sync_copy(d2d.at[pl.ds(row0, SR)], dst_v)

                def jloop(j, c1):
                    ca = pltpu.async_copy(el.at[src_v.at[j]], elg, sem_a)
                    cb = pltpu.async_copy(er.at[dst_v.at[j]], erg, sem_b)
                    ca.wait()
                    cb.wait()

                    def vloop(i, c2):
                        ee = jnp.exp(_lrelu(elg[i, :] + erg[i, :]))
                        eev[i, :] = ee
                        return c2

                    lax.fori_loop(0, 128, vloop, 0)
                    pltpu.sync_copy(eev, den_s.at[dst_v.at[j]], add=True)
                    return c1

                lax.fori_loop(0, SR, jloop, 0)
                return carry

            lax.fori_loop(0, NSTRIP, strip, 0)
            plsc.subcore_barrier()
            pltpu.sync_copy(den_s.at[pl.ds(tid * zrows, zrows)],
                            den_o.at[pl.ds(tid * zrows, zrows)])

        rel_args = (
            (el0, er0, (s0, d0), den0),
            (el1, er1, (s1, d1), den1),
            (el2, er2, (s2, d2), den2),
        )
        for r in range(3):
            el, er, (s2d, d2d), den_o = rel_args[r]
            pl.when(cid == r % 2)(
                functools.partial(do_rel, el, er, s2d, d2d, den_o))

    return run(*elps, *erps, z16, *edges)


def _messages(feats, elps, erps, dens, edges, binit):
    mesh = plsc.VectorSubcoreMesh(core_axis_name="c", subcore_axis_name="s")

    @functools.partial(
        pl.kernel,
        out_type=jax.ShapeDtypeStruct((NOUT, F), jnp.float32),
        mesh=mesh,
        compiler_params=pltpu.CompilerParams(use_tc_tiling_on_sc=False, needs_layout_passes=False),
        scratch_types=[
            pltpu.VMEM_SHARED((CS + 8, F), jnp.float32),
            pltpu.VMEM((SR, 128), jnp.int32),
            pltpu.VMEM((SR, 128), jnp.int32),
            pltpu.VMEM((EPT + BB,), jnp.int32),
            pltpu.VMEM((BB, F), jnp.float32),
            pltpu.VMEM((BB, W16), jnp.float32),
            pltpu.VMEM((BB, W16), jnp.float32),
            pltpu.VMEM((BB, W16), jnp.float32),
            pltpu.VMEM((1, 128), jnp.int32),
            pltpu.VMEM((1, 128), jnp.int32),
            pltpu.VMEM((1, 128), jnp.int32),
            pltpu.SemaphoreType.DMA,
        ],
    )
    def run(f0, f1, f2, el0, el1, el2, er0, er1, er2,
            den0, den1, den2, s0, d0, s1, d1, s2, d2, binit_hbm,
            out,
            acc, src_v, dst_v, sel_sd, fbuf,
            elg, erg, deng, isrc2, idst2a, idst2r, sem):
        cid = lax.axis_index("c")
        tid = lax.axis_index("s")
        rels = ((f0, el0, er0, den0, s0, d0),
                (f1, el1, er1, den1, s1, d1),
                (f2, el2, er2, den2, s2, d2))
        trash16 = jnp.full((16,), CS << 17, jnp.int32)

        def chunk(k, carry):
            lo = cid * CR + k * CS
            pltpu.sync_copy(binit_hbm, acc.at[pl.ds(tid * WB, WB)])
            plsc.subcore_barrier()
            for r in range(3):
                feat, elp, erp, den, s2d, d2d = rels[r]

                def strip(st, cur):
                    row0 = tid * TROWS + st * SR
                    pltpu.sync_copy(s2d.at[pl.ds(row0, SR)], src_v)
                    pltpu.sync_copy(d2d.at[pl.ds(row0, SR)], dst_v)

                    def scan_body(i, cur_v):
                        row = i >> 3
                        col = (i & 7) << 4
                        s = src_v[row, pl.ds(col, 16)]
                        d = dst_v[row, pl.ds(col, 16)]
                        m = (d >= lo) & (d < lo + CS)
                        mi = m.astype(jnp.int32)
                        cs = plsc.cumsum(mi)
                        pos = cur_v + cs - 1
                        packed = s | ((d - lo) << 17)
                        plsc.store_scatter(sel_sd, [pos], packed, mask=m)
                        return cur_v + _splat(cs, 15)

                    return lax.fori_loop(0, SR * 8, scan_body, cur)

                cur_v = lax.fori_loop(0, NSTRIP, strip,
                                      jnp.zeros((16,), jnp.int32))
                cnt = jnp.sum(cur_v) >> 4
                for q in range(BB // 16):
                    sel_sd[pl.ds(cnt + q * 16, 16)] = trash16
                nb = (cnt + (BB - 1)) >> 7

                def batch_body(b, carry2):
                    bo = b * BB
                    for q in range(BB // 16):
                        pv = sel_sd[pl.ds(bo + q * 16, 16)]
                        dv = pv >> 17
                        isrc2[0, pl.ds(q * 16, 16)] = pv & 0x1FFFF
                        idst2r[0, pl.ds(q * 16, 16)] = dv
                        idst2a[0, pl.ds(q * 16, 16)] = jnp.minimum(
                            dv + lo, NPAD - 8)
                    c1 = pltpu.async_copy(feat.at[isrc2.at[0]], fbuf, sem)
                    c2 = pltpu.async_copy(elp.at[isrc2.at[0]], elg, sem)
                    c3 = pltpu.async_copy(erp.at[idst2a.at[0]], erg, sem)
                    c4 = pltpu.async_copy(den.at[idst2a.at[0]], deng, sem)
                    c1.wait()
                    c2.wait()
                    c3.wait()
                    c4.wait()

                    def mloop(eb, c2_):
                        for u in range(4):
                            e = eb * 4 + u
                            ee = jnp.exp(_lrelu(elg[e, :] + erg[e, :]))
                            a = ee / jnp.maximum(deng[e, :],
                                                 jnp.float32(1e-9))
                            for h in range(H):
                                sp = _splat(a, h)
                                for p2 in range(2):
                                    p = h * 2 + p2
                                    fv = fbuf[e, pl.ds(p * 16, 16)]
                                    fbuf[e, pl.ds(p * 16, 16)] = fv * sp
                        return c2_

                    lax.fori_loop(0, BB // 4, mloop, 0)
                    pltpu.sync_copy(fbuf, acc.at[idst2r.at[0]], add=True)
                    return carry2

                lax.fori_loop(0, nb, batch_body, 0)
            plsc.subcore_barrier()
            pltpu.sync_copy(acc.at[pl.ds(tid * WB, WB)],
                            out.at[pl.ds(lo + tid * WB, WB)])
            return carry

        lax.fori_loop(0, NK, chunk, 0)

    return run(*feats, *elps, *erps, *dens, *edges, binit)


def _pad_nodes(a):
    return jnp.pad(a, ((0, NPAD - N), (0, 0)))


def _pad_edges(ei):
    s = jnp.concatenate([ei[0], jnp.zeros((E_PAD - E,), jnp.int32)])
    d = jnp.concatenate([ei[1], jnp.full((E_PAD - E,), N, jnp.int32)])
    return s.reshape(EROWS, 128), d.reshape(EROWS, 128)


def kernel(x, edge_index_r0, edge_index_r1, edge_index_r2,
           W_r0, attn_l_r0, attn_r_r0, bias_r0,
           W_r1, attn_l_r1, attn_r_r1, bias_r1,
           W_r2, attn_l_r2, attn_r_r2, bias_r2):
    Ws = jnp.stack([W_r0, W_r1, W_r2])
    ALs = jnp.stack([_expand_attn(attn_l_r0), _expand_attn(attn_l_r1),
                     _expand_attn(attn_l_r2)])
    ARs = jnp.stack([_expand_attn(attn_r_r0), _expand_attn(attn_r_r1),
                     _expand_attn(attn_r_r2)])

    f0, f1, f2, el0, er0, el1, er1, el2, er2 = _dense(x, Ws, ALs, ARs)
    elps = [_pad_nodes(el0), _pad_nodes(el1), _pad_nodes(el2)]
    erps = [_pad_nodes(er0), _pad_nodes(er1), _pad_nodes(er2)]

    s0, d0 = _pad_edges(edge_index_r0)
    s1, d1 = _pad_edges(edge_index_r1)
    s2, d2 = _pad_edges(edge_index_r2)
    edges = (s0, d0, s1, d1, s2, d2)

    z16 = jnp.zeros((NPAD // NT, W16), jnp.float32)
    den0, den1, den2 = _denoms(elps, erps, edges, z16)

    bias_sum = (bias_r0 + bias_r1 + bias_r2).astype(jnp.float32)
    binit = jnp.tile(bias_sum.reshape(1, F), (WB, 1))
    out_full = _messages((f0, f1, f2), elps, erps, (den0, den1, den2),
                         edges, binit)
    return out_full[:N]


# speedup vs baseline: 1.2485x; 1.2427x over previous
import functools

import jax
import jax.numpy as jnp
from jax import lax
from jax.experimental import pallas as pl
from jax.experimental.pallas import tpu as pltpu
from jax.experimental.pallas import tpu_sc as plsc

N = 100000
E = 200000
IN = 128
H = 4
DH = 32
F = H * DH

NC = 2
NT = 16
W16 = 16

EPT = 13312
TROWS = EPT // 128
E_PAD = EPT * NT
EROWS = E_PAD // 128

NK = 8
CS = 6272
CR = NK * CS
NOUT = NC * CR
NPAD = NOUT
WB = CS // NT
BB = 128
SR = 8
NSTRIP = TROWS // SR

_SPLAT_DNUMS = lax.GatherDimensionNumbers(
    offset_dims=(), collapsed_slice_dims=(0,), start_index_map=(0,))


def _splat(v, h):
    idx = jnp.full((16, 1), h, dtype=jnp.int32)
    return lax.gather(v, idx, _SPLAT_DNUMS, (1,),
                      mode=lax.GatherScatterMode.PROMISE_IN_BOUNDS)


def _expand_attn(a):
    eye = jnp.eye(W16, dtype=a.dtype)[:H]
    return (a[:, :, None] * eye[:, None, :]).reshape(IN, W16)


def _dense(x, Ws, ALs, ARs):
    R = 1000

    def body(x_ref, w_ref, al_ref, ar_ref,
             f0, f1, f2, el0, er0, el1, er1, el2, er2):
        xb = x_ref[...]
        outs = ((f0, el0, er0), (f1, el1, er1), (f2, el2, er2))
        for r in range(3):
            fr, elr, err = outs[r]
            feat = jnp.dot(xb, w_ref[r], preferred_element_type=jnp.float32)
            fr[...] = feat
            elr[...] = jnp.dot(feat, al_ref[r],
                               preferred_element_type=jnp.float32)
            err[...] = jnp.dot(feat, ar_ref[r],
                               preferred_element_type=jnp.float32)

    return pl.pallas_call(
        body,
        grid=(N // R,),
        in_specs=[
            pl.BlockSpec((R, IN), lambda i: (i, 0)),
            pl.BlockSpec((3, IN, F), lambda i: (0, 0, 0)),
            pl.BlockSpec((3, IN, W16), lambda i: (0, 0, 0)),
            pl.BlockSpec((3, IN, W16), lambda i: (0, 0, 0)),
        ],
        out_specs=[pl.BlockSpec((R, F), lambda i: (i, 0))] * 3
        + [pl.BlockSpec((R, W16), lambda i: (i, 0))] * 6,
        out_shape=[jax.ShapeDtypeStruct((N, F), jnp.float32)] * 3
        + [jax.ShapeDtypeStruct((N, W16), jnp.float32)] * 6,
    )(x, Ws, ALs, ARs)


def _lrelu(v):
    return jnp.where(v >= 0.0, v, v * jnp.float32(0.2))


def _denoms(elps, erps, edges, z16):
    mesh = plsc.VectorSubcoreMesh(core_axis_name="c", subcore_axis_name="s")

    @functools.partial(
        pl.kernel,
        out_type=[jax.ShapeDtypeStruct((NPAD, W16), jnp.float32)] * 3,
        mesh=mesh,
        compiler_params=pltpu.CompilerParams(use_tc_tiling_on_sc=False, needs_layout_passes=False),
        scratch_types=[
            pltpu.VMEM_SHARED((NPAD, W16), jnp.float32),
            pltpu.VMEM((SR, 128), jnp.int32),
            pltpu.VMEM((SR, 128), jnp.int32),
            pltpu.VMEM((128, W16), jnp.float32),
            pltpu.VMEM((128, W16), jnp.float32),
            pltpu.VMEM((128, W16), jnp.float32),
            pltpu.SemaphoreType.DMA,
            pltpu.SemaphoreType.DMA,
        ],
    )
    def run(el0, el1, el2, er0, er1, er2, z_hbm,
            s0, d0, s1, d1, s2, d2,
            den0, den1, den2,
            den_s, src_v, dst_v, elg, erg, eev, sem_a, sem_b):
        cid = lax.axis_index("c")
        tid = lax.axis_index("s")
        zrows = NPAD // NT

        def do_rel(el, er, s2d, d2d, den_o):
            pltpu.sync_copy(z_hbm, den_s.at[pl.ds(tid * zrows, zrows)])
            plsc.subcore_barrier()

            def strip(st, carry):
                row0 = tid * TROWS + st * SR
                pltpu.sync_copy(s2d.at[pl.ds(row0, SR)], src_v)
                pltpu.sync_copy(d2d.at[pl.ds(row0, SR)], dst_v)

                def jloop(j, c1):
                    ca = pltpu.async_copy(el.at[src_v.at[j]], elg, sem_a)
                    cb = pltpu.async_copy(er.at[dst_v.at[j]], erg, sem_b)
                    ca.wait()
                    cb.wait()

                    def vloop(i, c2):
                        ee = jnp.exp(_lrelu(elg[i, :] + erg[i, :]))
                        eev[i, :] = ee
                        return c2

                    lax.fori_loop(0, 128, vloop, 0)
                    pltpu.sync_copy(eev, den_s.at[dst_v.at[j]], add=True)
                    return c1

                lax.fori_loop(0, SR, jloop, 0)
                return carry

            lax.fori_loop(0, NSTRIP, strip, 0)
            plsc.subcore_barrier()
            pltpu.sync_copy(den_s.at[pl.ds(tid * zrows, zrows)],
                            den_o.at[pl.ds(tid * zrows, zrows)])

        rel_args = (
            (el0, er0, (s0, d0), den0),
            (el1, er1, (s1, d1), den1),
            (el2, er2, (s2, d2), den2),
        )
        for r in range(3):
            el, er, (s2d, d2d), den_o = rel_args[r]
            pl.when(cid == r % 2)(
                functools.partial(do_rel, el, er, s2d, d2d, den_o))

    return run(*elps, *erps, z16, *edges)


def _messages(feats, elps, erps, dens, edges, binit):
    mesh = plsc.VectorSubcoreMesh(core_axis_name="c", subcore_axis_name="s")

    @functools.partial(
        pl.kernel,
        out_type=jax.ShapeDtypeStruct((NOUT, F), jnp.float32),
        mesh=mesh,
        compiler_params=pltpu.CompilerParams(use_tc_tiling_on_sc=False, needs_layout_passes=False),
        scratch_types=[
            pltpu.VMEM_SHARED((CS + 8, F), jnp.float32),
            pltpu.VMEM((SR, 128), jnp.int32),
            pltpu.VMEM((SR, 128), jnp.int32),
            pltpu.VMEM((EPT + BB,), jnp.int32),
            pltpu.VMEM((BB, F), jnp.float32),
            pltpu.VMEM((BB, W16), jnp.float32),
            pltpu.VMEM((BB, W16), jnp.float32),
            pltpu.VMEM((BB, W16), jnp.float32),
            pltpu.VMEM((1, 128), jnp.int32),
            pltpu.VMEM((1, 128), jnp.int32),
            pltpu.VMEM((1, 128), jnp.int32),
            pltpu.VMEM((BB, F), jnp.float32),
            pltpu.VMEM((BB, W16), jnp.float32),
            pltpu.VMEM((BB, W16), jnp.float32),
            pltpu.VMEM((BB, W16), jnp.float32),
            pltpu.VMEM((1, 128), jnp.int32),
            pltpu.VMEM((1, 128), jnp.int32),
            pltpu.VMEM((1, 128), jnp.int32),
            pltpu.SemaphoreType.DMA,
            pltpu.SemaphoreType.DMA,
        ],
    )
    def run(f0, f1, f2, el0, el1, el2, er0, er1, er2,
            den0, den1, den2, s0, d0, s1, d1, s2, d2, binit_hbm,
            out,
            acc, src_v, dst_v, sel_sd, fbuf,
            elg, erg, deng, isrc2, idst2a, idst2r,
            fbuf2, elg2, erg2, deng2, isrc2b, idst2ab, idst2rb,
            sem, sem2):
        cid = lax.axis_index("c")
        tid = lax.axis_index("s")
        rels = ((f0, el0, er0, den0, s0, d0),
                (f1, el1, er1, den1, s1, d1),
                (f2, el2, er2, den2, s2, d2))
        trash16 = jnp.full((16,), CS << 17, jnp.int32)

        def chunk(k, carry):
            lo = cid * CR + k * CS
            pltpu.sync_copy(binit_hbm, acc.at[pl.ds(tid * WB, WB)])
            plsc.subcore_barrier()
            for r in range(3):
                feat, elp, erp, den, s2d, d2d = rels[r]

                def strip(st, cur):
                    row0 = tid * TROWS + st * SR
                    pltpu.sync_copy(s2d.at[pl.ds(row0, SR)], src_v)
                    pltpu.sync_copy(d2d.at[pl.ds(row0, SR)], dst_v)

                    def scan_body(i, cur_v):
                        row = i >> 3
                        col = (i & 7) << 4
                        s = src_v[row, pl.ds(col, 16)]
                        d = dst_v[row, pl.ds(col, 16)]
                        m = (d >= lo) & (d < lo + CS)
                        mi = m.astype(jnp.int32)
                        cs = plsc.cumsum(mi)
                        pos = cur_v + cs - 1
                        packed = s | ((d - lo) << 17)
                        plsc.store_scatter(sel_sd, [pos], packed, mask=m)
                        return cur_v + _splat(cs, 15)

                    return lax.fori_loop(0, SR * 8, scan_body, cur)

                cur_v = lax.fori_loop(0, NSTRIP, strip,
                                      jnp.zeros((16,), jnp.int32))
                cnt = jnp.sum(cur_v) >> 4
                for q in range(BB // 16):
                    sel_sd[pl.ds(cnt + q * 16, 16)] = trash16
                nb = (cnt + (BB - 1)) >> 7
                bufs = ((fbuf, elg, erg, deng, isrc2, idst2a, idst2r, sem),
                        (fbuf2, elg2, erg2, deng2, isrc2b, idst2ab,
                         idst2rb, sem2))

                def fire(b, bi):
                    fb, eg, rg, dg, i2, ia, ir, sm = bufs[bi]
                    bo = b * BB
                    for q in range(BB // 16):
                        pv = sel_sd[pl.ds(bo + q * 16, 16)]
                        dv = pv >> 17
                        i2[0, pl.ds(q * 16, 16)] = pv & 0x1FFFF
                        ir[0, pl.ds(q * 16, 16)] = dv
                        ia[0, pl.ds(q * 16, 16)] = jnp.minimum(
                            dv + lo, NPAD - 8)
                    pltpu.async_copy(feat.at[i2.at[0]], fb, sm)
                    pltpu.async_copy(elp.at[i2.at[0]], eg, sm)
                    pltpu.async_copy(erp.at[ia.at[0]], rg, sm)
                    pltpu.async_copy(den.at[ia.at[0]], dg, sm)

                def drain(bi):
                    fb, eg, rg, dg, i2, ia, ir, sm = bufs[bi]
                    pltpu.make_async_copy(feat.at[i2.at[0]], fb, sm).wait()
                    pltpu.make_async_copy(elp.at[i2.at[0]], eg, sm).wait()
                    pltpu.make_async_copy(erp.at[ia.at[0]], rg, sm).wait()
                    pltpu.make_async_copy(den.at[ia.at[0]], dg, sm).wait()

                def consume(bi):
                    fb, eg, rg, dg, i2, ia, ir, sm = bufs[bi]

                    def mloop(eb, c2_):
                        for u in range(4):
                            e = eb * 4 + u
                            ee = jnp.exp(_lrelu(eg[e, :] + rg[e, :]))
                            a = ee / jnp.maximum(dg[e, :],
                                                 jnp.float32(1e-9))
                            for h in range(H):
                                sp = _splat(a, h)
                                for p2 in range(2):
                                    p = h * 2 + p2
                                    fv = fb[e, pl.ds(p * 16, 16)]
                                    fb[e, pl.ds(p * 16, 16)] = fv * sp
                        return c2_

                    lax.fori_loop(0, BB // 4, mloop, 0)
                    pltpu.sync_copy(fb, acc.at[ir.at[0]], add=True)

                pl.when(nb > 0)(lambda: fire(0, 0))

                def pair(p, carry2):
                    b1 = p * 2 + 1

                    def odd_fire():
                        fire(b1, 1)

                    pl.when(b1 < nb)(odd_fire)
                    drain(0)
                    consume(0)

                    def odd_consume():
                        pl.when(b1 + 1 < nb)(lambda: fire(b1 + 1, 0))
                        drain(1)
                        consume(1)

                    pl.when(b1 < nb)(odd_consume)
                    return carry2

                lax.fori_loop(0, (nb + 1) >> 1, pair, 0)
            plsc.subcore_barrier()
            pltpu.sync_copy(acc.at[pl.ds(tid * WB, WB)],
                            out.at[pl.ds(lo + tid * WB, WB)])
            return carry

        lax.fori_loop(0, NK, chunk, 0)

    return run(*feats, *elps, *erps, *dens, *edges, binit)


def _pad_nodes(a):
    return jnp.pad(a, ((0, NPAD - N), (0, 0)))


def _pad_edges(ei):
    s = jnp.concatenate([ei[0], jnp.zeros((E_PAD - E,), jnp.int32)])
    d = jnp.concatenate([ei[1], jnp.full((E_PAD - E,), N, jnp.int32)])
    return s.reshape(EROWS, 128), d.reshape(EROWS, 128)


def kernel(x, edge_index_r0, edge_index_r1, edge_index_r2,
           W_r0, attn_l_r0, attn_r_r0, bias_r0,
           W_r1, attn_l_r1, attn_r_r1, bias_r1,
           W_r2, attn_l_r2, attn_r_r2, bias_r2):
    Ws = jnp.stack([W_r0, W_r1, W_r2])
    ALs = jnp.stack([_expand_attn(attn_l_r0), _expand_attn(attn_l_r1),
                     _expand_attn(attn_l_r2)])
    ARs = jnp.stack([_expand_attn(attn_r_r0), _expand_attn(attn_r_r1),
                     _expand_attn(attn_r_r2)])

    f0, f1, f2, el0, er0, el1, er1, el2, er2 = _dense(x, Ws, ALs, ARs)
    elps = [_pad_nodes(el0), _pad_nodes(el1), _pad_nodes(el2)]
    erps = [_pad_nodes(er0), _pad_nodes(er1), _pad_nodes(er2)]

    s0, d0 = _pad_edges(edge_index_r0)
    s1, d1 = _pad_edges(edge_index_r1)
    s2, d2 = _pad_edges(edge_index_r2)
    edges = (s0, d0, s1, d1, s2, d2)

    z16 = jnp.zeros((NPAD // NT, W16), jnp.float32)
    den0, den1, den2 = _denoms(elps, erps, edges, z16)

    bias_sum = (bias_r0 + bias_r1 + bias_r2).astype(jnp.float32)
    binit = jnp.tile(bias_sum.reshape(1, F), (WB, 1))
    out_full = _messages((f0, f1, f2), elps, erps, (den0, den1, den2),
                         edges, binit)
    return out_full[:N]
